# Initial kernel scaffold; baseline (speedup 1.0000x reference)
import jax
import jax.numpy as jnp
from jax.experimental import pallas as pl


def kernel(src, tgt, src_coords, tgt_coords, W_in, b_in, kernel_points, kernel_weights, W_out, b_out, gamma, beta):
    raise NotImplementedError("write your pallas kernel here")


# speedup vs baseline: 4.2597x; 4.2597x over previous
import functools

import jax
import jax.numpy as jnp
from jax import lax
from jax.experimental import pallas as pl
from jax.experimental.pallas import tpu as pltpu
from jax.experimental.pallas import tpu_sc as plsc

N = 10000
C_IN = 256
C_MID = 128
C_OUT = 256
K = 15
M = 32
EXT = 0.1 * 2.0 / 2.5
SLOPE = 0.1

BR = 128
NPAD = 10112
NBLK = NPAD // BR
BR2 = 80
NBLK2 = N // BR2

NWORK = 32
CH = 128
BG = 327680
BPW = BG // NWORK
NCH = BPW // CH

BIGV = 1e30
BIGI = 1e9


def _fold_body(kw_ref, wo_ref, o_ref):
    o_ref[...] = jnp.dot(kw_ref[0], wo_ref[...],
                         preferred_element_type=jnp.float32)[None]


def _fold_weights(kernel_weights, W_out):
    return pl.pallas_call(
        _fold_body,
        grid=(K,),
        in_specs=[
            pl.BlockSpec((1, C_MID, C_MID), lambda k: (k, 0, 0)),
            pl.BlockSpec((C_MID, C_OUT), lambda k: (0, 0)),
        ],
        out_specs=pl.BlockSpec((1, C_MID, C_OUT), lambda k: (k, 0, 0)),
        out_shape=jax.ShapeDtypeStruct((K, C_MID, C_OUT), jnp.float32),
    )(kernel_weights, W_out)


def _lin_body(x_ref, w_ref, b_ref, o_ref):
    o_ref[...] = jnp.dot(x_ref[...], w_ref[...],
                         preferred_element_type=jnp.float32) + b_ref[0:1, :]


def _linear_in(x, W_in, b8):
    return pl.pallas_call(
        _lin_body,
        grid=(NBLK2,),
        in_specs=[
            pl.BlockSpec((BR2, C_IN), lambda i: (i, 0)),
            pl.BlockSpec((C_IN, C_MID), lambda i: (0, 0)),
            pl.BlockSpec((8, C_MID), lambda i: (0, 0)),
        ],
        out_specs=pl.BlockSpec((BR2, C_MID), lambda i: (i, 0)),
        out_shape=jax.ShapeDtypeStruct((N, C_MID), jnp.float32),
    )(x, W_in, b8)


def _topk_body(cb_ref, ct_ref, o_ref, d2_ref, jacc_ref):
    cb = cb_ref[...]
    ct = ct_ref[...]
    sqb = jnp.sum(cb * cb, axis=1, keepdims=True)
    sqa = jnp.sum(ct * ct, axis=0, keepdims=True)
    dot = jnp.dot(cb, ct, preferred_element_type=jnp.float32)
    d2 = sqb + sqa - 2.0 * dot
    colf = lax.broadcasted_iota(jnp.int32, (BR, NPAD), 1).astype(jnp.float32)
    d2_ref[...] = jnp.where(colf < float(N), d2, BIGV)
    jacc_ref[...] = jnp.zeros((BR, 128), jnp.float32)
    lane = lax.broadcasted_iota(jnp.int32, (BR, 128), 1)

    def body(it, _):
        d2c = d2_ref[...]
        m = jnp.min(d2c, axis=1, keepdims=True)
        cand = jnp.where(d2c <= m, colf, BIGI)
        j = jnp.min(cand, axis=1, keepdims=True)
        d2_ref[...] = jnp.where(cand == j, BIGV, d2c)
        jacc_ref[...] = jnp.where(lane == it,
                                  jnp.broadcast_to(j, (BR, 128)),
                                  jacc_ref[...])
        return 0

    lax.fori_loop(0, M, body, 0)
    o_ref[...] = jacc_ref[...].astype(jnp.int32)


def _topk_idx(cpad, ct):
    return pl.pallas_call(
        _topk_body,
        grid=(NBLK,),
        in_specs=[
            pl.BlockSpec((BR, 8), lambda i: (i, 0)),
            pl.BlockSpec((8, NPAD), lambda i: (0, 0)),
        ],
        out_specs=pl.BlockSpec((BR, 128), lambda i: (i, 0)),
        out_shape=jax.ShapeDtypeStruct((NPAD, 128), jnp.int32),
        scratch_shapes=[
            pltpu.VMEM((BR, NPAD), jnp.float32),
            pltpu.VMEM((BR, 128), jnp.float32),
        ],
    )(cpad, ct)


def _sc_gather_body(idx_hbm, ft_hbm, ct_hbm, of_hbm, oc_hbm,
                    idx_v, rf_v, rc_v, s1, s2):
    wid = lax.axis_index("s") * 2 + lax.axis_index("c")

    def step(ch, carry):
        base = pl.multiple_of(wid * BPW + ch * CH, CH)
        pltpu.sync_copy(idx_hbm.at[pl.ds(base, CH)], idx_v)
        cp1 = pltpu.async_copy(ft_hbm.at[idx_v], rf_v, s1)
        cp2 = pltpu.async_copy(ct_hbm.at[idx_v], rc_v, s2)
        cp1.wait()
        cp2.wait()
        pltpu.sync_copy(rf_v, of_hbm.at[pl.ds(base, CH)])
        pltpu.sync_copy(rc_v, oc_hbm.at[pl.ds(base, CH)])
        return carry

    lax.fori_loop(0, NCH, step, 0)


def _sc_gather(idxp, feats, c16):
    mesh = plsc.VectorSubcoreMesh(core_axis_name="c", subcore_axis_name="s")
    fn = functools.partial(
        pl.kernel,
        mesh=mesh,
        out_type=(
            jax.ShapeDtypeStruct((BG, C_MID), jnp.float32),
            jax.ShapeDtypeStruct((BG, 128), jnp.float32),
        ),
        scratch_types=[
            pltpu.VMEM((CH,), jnp.int32),
            pltpu.VMEM((CH, C_MID), jnp.float32),
            pltpu.VMEM((CH, 128), jnp.float32),
            pltpu.SemaphoreType.DMA,
            pltpu.SemaphoreType.DMA,
        ],
    )(_sc_gather_body)
    return fn(idxp, feats, c16)


def _agg_body(nf_ref, nc_ref, cb_ref, kp_ref, wp_ref, bo_ref, o_ref, st_ref):
    i = pl.program_id(0)
    nf = nf_ref[...].reshape(BR2 * M, C_MID)
    nc = nc_ref[...].reshape(BR2 * M, 128)
    cb = cb_ref[...]
    cbr = jnp.broadcast_to(cb[:, None, :], (BR2, M, 128)).reshape(BR2 * M, 128)
    y = nc - cbr
    kpt = kp_ref[...]
    yy = jnp.sum(y * y, axis=1, keepdims=True)
    yk = jnp.dot(y, kpt, preferred_element_type=jnp.float32)
    kpsq = jnp.sum(kpt * kpt, axis=0, keepdims=True)
    dsq = jnp.maximum(yy - 2.0 * yk + kpsq, 0.0)
    dist = jnp.sqrt(dsq + 1e-12)
    h = jnp.maximum(0.0, 1.0 - dist / EXT)
    parts = []
    for k in range(K):
        wk = h[:, k:k + 1] * nf
        parts.append(jnp.sum(wk.reshape(BR2, M, C_MID), axis=1))
    agg = jnp.concatenate(parts, axis=1)
    ob = jnp.dot(agg, wp_ref[...],
                 preferred_element_type=jnp.float32) + bo_ref[0:1, :]
    o_ref[...] = ob
    colsum = jnp.sum(ob, axis=0, keepdims=True)
    colsq = jnp.sum(ob * ob, axis=0, keepdims=True)
    upd = jnp.concatenate(
        [colsum, colsq, jnp.zeros((6, C_OUT), jnp.float32)], axis=0)

    @pl.when(i == 0)
    def _():
        st_ref[...] = jnp.zeros((8, C_OUT), jnp.float32)

    st_ref[...] += upd


def _aggregate(nf3, nc3, c16, kpt, wp2, bo8):
    return pl.pallas_call(
        _agg_body,
        grid=(NBLK2,),
        in_specs=[
            pl.BlockSpec((BR2, M, C_MID), lambda i: (i, 0, 0)),
            pl.BlockSpec((BR2, M, 128), lambda i: (i, 0, 0)),
            pl.BlockSpec((BR2, 128), lambda i: (i, 0)),
            pl.BlockSpec((128, 128), lambda i: (0, 0)),
            pl.BlockSpec((K * C_MID, C_OUT), lambda i: (0, 0)),
            pl.BlockSpec((8, C_OUT), lambda i: (0, 0)),
        ],
        out_specs=(
            pl.BlockSpec((BR2, C_OUT), lambda i: (i, 0)),
            pl.BlockSpec((8, C_OUT), lambda i: (0, 0)),
        ),
        out_shape=(
            jax.ShapeDtypeStruct((N, C_OUT), jnp.float32),
            jax.ShapeDtypeStruct((8, C_OUT), jnp.float32),
        ),
    )(nf3, nc3, c16, kpt, wp2, bo8)


def _bn_body(x_ref, st_ref, gb_ref, o_ref):
    st = st_ref[...]
    mu = st[0:1, :] * (1.0 / N)
    ex2 = st[1:2, :] * (1.0 / N)
    var = ex2 - mu * mu
    scale = gb_ref[0:1, :] / jnp.sqrt(var + 1e-5)
    y = (x_ref[...] - mu) * scale + gb_ref[1:2, :]
    o_ref[...] = jnp.where(y >= 0.0, y, SLOPE * y)


def _bn_leaky(x, st, gb):
    return pl.pallas_call(
        _bn_body,
        grid=(NBLK2,),
        in_specs=[
            pl.BlockSpec((BR2, C_OUT), lambda i: (i, 0)),
            pl.BlockSpec((8, C_OUT), lambda i: (0, 0)),
            pl.BlockSpec((8, C_OUT), lambda i: (0, 0)),
        ],
        out_specs=pl.BlockSpec((BR2, C_OUT), lambda i: (i, 0)),
        out_shape=jax.ShapeDtypeStruct((N, C_OUT), jnp.float32),
    )(x, st, gb)


def _pad_rows8(v):
    return jnp.pad(v[None, :].astype(jnp.float32), ((0, 7), (0, 0)))


def _one_cloud(x, coords, W_in, b8, kpt, wp2, bo8, gb):
    feats = _linear_in(x, W_in, b8)
    cpad = jnp.pad(coords, ((0, NPAD - N), (0, 5)))
    ct = cpad.T
    idx_full = _topk_idx(cpad, ct)
    idx = idx_full[:N, :M].reshape(N * M)
    idxp = jnp.pad(idx, (0, BG - N * M))
    c128 = jnp.pad(coords, ((0, 0), (0, 125)))
    gf, gc = _sc_gather(idxp, feats, c128)
    nf3 = gf.reshape(BG // M, M, C_MID)
    nc3 = gc.reshape(BG // M, M, 128)
    out2, st = _aggregate(nf3, nc3, c128, kpt, wp2, bo8)
    return _bn_leaky(out2, st, gb)


def kernel(src, tgt, src_coords, tgt_coords, W_in, b_in, kernel_points,
           kernel_weights, W_out, b_out, gamma, beta):
    wp2 = _fold_weights(kernel_weights, W_out).reshape(K * C_MID, C_OUT)
    b8 = _pad_rows8(b_in)
    bo8 = _pad_rows8(b_out)
    gb = jnp.concatenate([_pad_rows8(gamma)[0:1], _pad_rows8(beta)[0:1],
                          jnp.zeros((6, C_OUT), jnp.float32)], axis=0)
    kpt = jnp.pad(kernel_points.T, ((0, 125), (0, 128 - K)))
    s = _one_cloud(src, src_coords, W_in, b8, kpt, wp2, bo8, gb)
    t = _one_cloud(tgt, tgt_coords, W_in, b8, kpt, wp2, bo8, gb)
    return (s, t, src_coords, tgt_coords)
